# Initial kernel scaffold; baseline (speedup 1.0000x reference)
import jax
import jax.numpy as jnp
from jax.experimental import pallas as pl


def kernel(x_query, vis_mark, P, task_keys):
    raise NotImplementedError("write your pallas kernel here")


# speedup vs baseline: 3.4911x; 3.4911x over previous
import jax
import jax.numpy as jnp
from jax.experimental import pallas as pl
from jax.experimental.pallas import tpu as pltpu

_B = 256
_NL = 12
_KD = 768
_NT = 10
_NP = 8
_ED = 768
_NK = 100
_PD = _NP * _ED


def _layer_body(q_ref, keys_ref, p_ref, out_ref):
    q = q_ref[0]
    keys = keys_ref[...]
    ptab = p_ref[0]
    knorm = jax.lax.dot_general(
        jnp.ones((1, _KD), jnp.float32), keys * keys,
        (((1,), (1,)), ((), ())), preferred_element_type=jnp.float32)
    cross = jax.lax.dot_general(
        q, keys, (((1,), (1,)), ((), ())),
        preferred_element_type=jnp.float32)
    scores = knorm - 2.0 * cross
    colidx = jax.lax.broadcasted_iota(jnp.int32, (_B, _NK), 1)
    mval = jnp.min(scores, axis=1, keepdims=True)
    idx = jnp.min(jnp.where(scores == mval, colidx, _NK), axis=1, keepdims=True)
    task = idx // _NK
    onehot = (task == jax.lax.broadcasted_iota(jnp.int32, (_B, _NT), 1)
              ).astype(jnp.float32)
    out_ref[0] = jax.lax.dot_general(
        onehot, ptab, (((1,), (0,)), ((), ())),
        preferred_element_type=jnp.float32)


def kernel(x_query, vis_mark, P, task_keys):
    del vis_mark
    xq = jnp.transpose(x_query, (1, 0, 2))
    p2 = P.reshape(_NL, _NT, _PD)
    out = pl.pallas_call(
        _layer_body,
        grid=(_NL,),
        in_specs=[
            pl.BlockSpec((1, _B, _KD), lambda l: (l, 0, 0)),
            pl.BlockSpec((_NK, _KD), lambda l: (0, 0)),
            pl.BlockSpec((1, _NT, _PD), lambda l: (l, 0, 0)),
        ],
        out_specs=pl.BlockSpec((1, _B, _PD), lambda l: (l, 0, 0)),
        out_shape=jax.ShapeDtypeStruct((_NL, _B, _PD), jnp.float32),
    )(xq, task_keys, p2)
    return (out.reshape(_NL, _B, _NP, _ED), jnp.float32(0.0))


# speedup vs baseline: 7.4998x; 2.1483x over previous
import jax
import jax.numpy as jnp
from jax.experimental import pallas as pl
from jax.experimental.pallas import tpu as pltpu

_B = 256
_NL = 12
_KD = 768
_NT = 10
_NP = 8
_ED = 768
_NK = 100
_PD = _NP * _ED


def _layer_body(q_ref, keys_ref, p_ref, out_ref):
    q = q_ref[0]
    keys = keys_ref[...]
    knorm = jax.lax.dot_general(
        jnp.ones((1, _KD), jnp.float32), keys * keys,
        (((1,), (1,)), ((), ())), preferred_element_type=jnp.float32)
    cross = jax.lax.dot_general(
        q, keys, (((1,), (1,)), ((), ())),
        preferred_element_type=jnp.float32)
    scores = knorm - 2.0 * cross
    colidx = jax.lax.broadcasted_iota(jnp.int32, (_B, _NK), 1)
    mval = jnp.min(scores, axis=1, keepdims=True)
    idx = jnp.min(jnp.where(scores == mval, colidx, _NK), axis=1, keepdims=True)
    task = idx // _NK
    onehot = (task == jax.lax.broadcasted_iota(jnp.int32, (_B, _NT), 1)
              ).astype(jnp.float32)
    for p in range(_NP):
        out_ref[0, :, p, :] = jax.lax.dot_general(
            onehot, p_ref[0, :, p, :], (((1,), (0,)), ((), ())),
            preferred_element_type=jnp.float32)


def kernel(x_query, vis_mark, P, task_keys):
    del vis_mark
    xq = jnp.transpose(x_query, (1, 0, 2))
    out = pl.pallas_call(
        _layer_body,
        grid=(_NL,),
        in_specs=[
            pl.BlockSpec((1, _B, _KD), lambda l: (l, 0, 0)),
            pl.BlockSpec((_NK, _KD), lambda l: (0, 0)),
            pl.BlockSpec((1, _NT, _NP, _ED), lambda l: (l, 0, 0, 0)),
        ],
        out_specs=pl.BlockSpec((1, _B, _NP, _ED), lambda l: (l, 0, 0, 0)),
        out_shape=jax.ShapeDtypeStruct((_NL, _B, _NP, _ED), jnp.float32),
    )(xq, task_keys, P)
    return (out, jnp.float32(0.0))
